# Initial kernel scaffold; baseline (speedup 1.0000x reference)
import jax
import jax.numpy as jnp
from jax.experimental import pallas as pl


def kernel(x, pos_table):
    raise NotImplementedError("write your pallas kernel here")


# speedup vs baseline: 3.2860x; 3.2860x over previous
import jax
import jax.numpy as jnp
from jax.experimental import pallas as pl


_S_BLK = 512


def _add_body(x_ref, p_ref, o_ref):
    o_ref[...] = x_ref[...] + p_ref[...][None, :, :]


def kernel(x, pos_table):
    B, S, F = x.shape
    grid = (S // _S_BLK,)
    return pl.pallas_call(
        _add_body,
        grid=grid,
        in_specs=[
            pl.BlockSpec((B, _S_BLK, F), lambda i: (0, i, 0)),
            pl.BlockSpec((_S_BLK, F), lambda i: (i, 0)),
        ],
        out_specs=pl.BlockSpec((B, _S_BLK, F), lambda i: (0, i, 0)),
        out_shape=jax.ShapeDtypeStruct((B, S, F), x.dtype),
    )(x, pos_table)
